# baseline (speedup 1.0000x reference)
import functools

import jax
import jax.numpy as jnp
from jax import lax
from jax.experimental import pallas as pl
from jax.experimental.pallas import tpu as pltpu
from jax.experimental.pallas import tpu_sc as plsc

_BETA = 0.25
_M_TILE = 2048
_K_CHUNK = 2048
_PAD_VAL = 3.0e4

_NC = 2
_NS = 16
_NW = _NC * _NS


def _vq_body(z_ref, cb_ref, codes_ref, loss_ref, best_rnd,
             best_exact, best_idx, *, k_chunk, inv_norm):
    i = pl.program_id(0)
    j = pl.program_id(1)
    ni = pl.num_programs(0)
    nj = pl.num_programs(1)

    z = z_ref[...]
    cb = cb_ref[...]
    ndot = lax.dot_general(z * (-2.0), cb, (((1,), (1,)), ((), ())),
                           preferred_element_type=jnp.float32)
    z_sq = jnp.sum(z * z, axis=1, keepdims=True)
    e_sq = jnp.sum(cb * cb, axis=1)[None, :]
    dists = (z_sq + ndot) + e_sq

    v = dists
    idx = lax.broadcasted_iota(jnp.int32, dists.shape, 1) + j * k_chunk
    w = v.shape[1]
    while w > 128:
        h = w // 2
        a, b_ = v[:, :h], v[:, h:]
        ia, ib = idx[:, :h], idx[:, h:]
        t = b_ < a
        v = jnp.where(t, b_, a)
        idx = jnp.where(t, ib, ia)
        w = h
    local_min = jnp.min(v, axis=1, keepdims=True)
    masked = jnp.where(v == local_min, idx, jnp.int32(2**30))
    local_arg = jnp.min(masked, axis=1, keepdims=True)
    local_rnd = local_min.astype(jnp.bfloat16).astype(jnp.float32)

    @pl.when(j == 0)
    def _():
        best_rnd[...] = local_rnd
        best_exact[...] = local_min
        best_idx[...] = local_arg

    @pl.when(j > 0)
    def _():
        br = best_rnd[...]
        better = local_min < br
        best_rnd[...] = jnp.where(better, local_rnd, br)
        be = jnp.where(better, local_min, best_exact[...])
        best_exact[...] = be
        bi = jnp.where(better, local_arg, best_idx[...])
        best_idx[...] = bi

        @pl.when(j == nj - 1)
        def _():
            codes_ref[...] = bi
            part = jnp.sum(be, keepdims=True).reshape(1, 1)
            prev = jnp.where(i == 0, jnp.zeros((1, 1), jnp.float32),
                             loss_ref[...])
            total = prev + part
            loss_ref[...] = jnp.where(i == ni - 1, total * inv_norm, total)


def _distance_argmin(z_flat, cb_padded, inv_norm):
    bt, d = z_flat.shape
    kp = cb_padded.shape[0]
    nj = kp // _K_CHUNK
    grid = (bt // _M_TILE, nj)
    codes, loss = pl.pallas_call(
        functools.partial(_vq_body, k_chunk=_K_CHUNK, inv_norm=inv_norm),
        grid=grid,
        in_specs=[
            pl.BlockSpec((_M_TILE, d), lambda i, j: (i, 0)),
            pl.BlockSpec((_K_CHUNK, d), lambda i, j: (j, 0)),
        ],
        out_specs=[
            pl.BlockSpec((_M_TILE, 1), lambda i, j: (i, 0)),
            pl.BlockSpec((1, 1), lambda i, j: (0, 0)),
        ],
        out_shape=[
            jax.ShapeDtypeStruct((bt, 1), jnp.int32),
            jax.ShapeDtypeStruct((1, 1), jnp.float32),
        ],
        scratch_shapes=[
            pltpu.VMEM((_M_TILE, 1), jnp.float32),
            pltpu.VMEM((_M_TILE, 1), jnp.float32),
            pltpu.VMEM((_M_TILE, 1), jnp.int32),
        ],
        compiler_params=pltpu.CompilerParams(
            dimension_semantics=("arbitrary", "arbitrary")),
    )(z_flat, cb_padded)
    return codes, loss


def _make_sc_gather(v, d, b):
    b_per_w = b // _NW
    mesh = plsc.VectorSubcoreMesh(core_axis_name="c", subcore_axis_name="s")

    @functools.partial(
        pl.kernel, mesh=mesh,
        out_type=jax.ShapeDtypeStruct((b, d), jnp.float32),
        scratch_types=[
            pltpu.VMEM((b_per_w,), jnp.int32),
            pltpu.VMEM((b_per_w, d), jnp.float32),
            pltpu.SemaphoreType.DMA,
        ],
    )
    def gather(table_hbm, idx_hbm, out_hbm, idx_v, rows_v, sem):
        wid = lax.axis_index("s") * _NC + lax.axis_index("c")
        base = wid * b_per_w
        pltpu.sync_copy(idx_hbm.at[pl.ds(base, b_per_w)], idx_v)
        pltpu.async_copy(table_hbm.at[idx_v], rows_v, sem).wait()
        pltpu.sync_copy(rows_v, out_hbm.at[pl.ds(base, b_per_w)])

    return gather


def kernel(z, codebook):
    b, t, d = z.shape
    k = codebook.shape[0]
    z_flat = z.reshape(-1, d)
    bt = b * t
    n_chunks = -(-k // _K_CHUNK)
    kp = n_chunks * _K_CHUNK
    if kp != k:
        codebook_p = jnp.concatenate(
            [codebook, jnp.full((kp - k, d), _PAD_VAL, jnp.float32)], axis=0)
    else:
        codebook_p = codebook
    half = bt // 2
    inv_norm = (1.0 + _BETA) / float(bt * d)
    gather = _make_sc_gather(k, d, half)
    codes0, loss0 = _distance_argmin(z_flat[:half], codebook_p, inv_norm)
    zq0 = gather(codebook, codes0.reshape(-1))
    codes1, loss1 = _distance_argmin(z_flat[half:], codebook_p, inv_norm)
    zq1 = gather(codebook, codes1.reshape(-1))
    codes_flat = jnp.concatenate([codes0.reshape(-1), codes1.reshape(-1)])
    zq_flat = jnp.concatenate([zq0, zq1], axis=0)
    z_q_st = zq_flat.reshape(b, t, d)
    codes = codes_flat.reshape(b, t)
    vq_loss = loss0[0, 0] + loss1[0, 0]
    return z_q_st, codes, vq_loss


# speedup vs baseline: 1.1646x; 1.1646x over previous
import functools

import jax
import jax.numpy as jnp
from jax import lax
from jax.experimental import pallas as pl
from jax.experimental.pallas import tpu as pltpu
from jax.experimental.pallas import tpu_sc as plsc

_BETA = 0.25
_M_TILE = 2048
_K_CHUNK = 2048
_PAD_VAL = 3.0e4

_NC = 2
_NS = 16
_NW = _NC * _NS


def _vq_body(z_ref, cb_ref, codes_ref, loss_ref, best_rnd,
             best_exact, best_idx, *, k_chunk, inv_norm):
    i = pl.program_id(0)
    j = pl.program_id(1)
    ni = pl.num_programs(0)
    nj = pl.num_programs(1)

    z = z_ref[...]
    cb = cb_ref[...]
    ndot = lax.dot_general(z * (-2.0), cb, (((1,), (1,)), ((), ())),
                           preferred_element_type=jnp.float32)
    z_sq = jnp.sum(z * z, axis=1, keepdims=True)
    e_sq = jnp.sum(cb * cb, axis=1)[None, :]
    dists = (z_sq + ndot) + e_sq

    v = dists
    idx = lax.broadcasted_iota(jnp.int32, dists.shape, 1) + j * k_chunk
    w = v.shape[1]
    while w > 128:
        h = w // 2
        a, b_ = v[:, :h], v[:, h:]
        ia, ib = idx[:, :h], idx[:, h:]
        t = b_ < a
        v = jnp.where(t, b_, a)
        idx = jnp.where(t, ib, ia)
        w = h
    local_min = jnp.min(v, axis=1, keepdims=True)
    masked = jnp.where(v == local_min, idx, jnp.int32(2**30))
    local_arg = jnp.min(masked, axis=1, keepdims=True)
    local_rnd = local_min.astype(jnp.bfloat16).astype(jnp.float32)

    @pl.when(j == 0)
    def _():
        best_rnd[...] = local_rnd
        best_exact[...] = local_min
        best_idx[...] = local_arg

    @pl.when(j > 0)
    def _():
        br = best_rnd[...]
        better = local_min < br
        best_rnd[...] = jnp.where(better, local_rnd, br)
        be = jnp.where(better, local_min, best_exact[...])
        best_exact[...] = be
        bi = jnp.where(better, local_arg, best_idx[...])
        best_idx[...] = bi

        @pl.when(j == nj - 1)
        def _():
            codes_ref[...] = bi
            part = jnp.sum(be, keepdims=True).reshape(1, 1)
            prev = jnp.where(i == 0, jnp.zeros((1, 1), jnp.float32),
                             loss_ref[...])
            total = prev + part
            loss_ref[...] = jnp.where(i == ni - 1, total * inv_norm, total)


def _distance_argmin(z_flat, cb_padded, inv_norm):
    bt, d = z_flat.shape
    kp = cb_padded.shape[0]
    nj = kp // _K_CHUNK
    grid = (bt // _M_TILE, nj)
    codes, loss = pl.pallas_call(
        functools.partial(_vq_body, k_chunk=_K_CHUNK, inv_norm=inv_norm),
        grid=grid,
        in_specs=[
            pl.BlockSpec((_M_TILE, d), lambda i, j: (i, 0)),
            pl.BlockSpec((_K_CHUNK, d), lambda i, j: (j, 0)),
        ],
        out_specs=[
            pl.BlockSpec((_M_TILE, 1), lambda i, j: (i, 0)),
            pl.BlockSpec((1, 1), lambda i, j: (0, 0)),
        ],
        out_shape=[
            jax.ShapeDtypeStruct((bt, 1), jnp.int32),
            jax.ShapeDtypeStruct((1, 1), jnp.float32),
        ],
        scratch_shapes=[
            pltpu.VMEM((_M_TILE, 1), jnp.float32),
            pltpu.VMEM((_M_TILE, 1), jnp.float32),
            pltpu.VMEM((_M_TILE, 1), jnp.int32),
        ],
        compiler_params=pltpu.CompilerParams(
            dimension_semantics=("arbitrary", "arbitrary")),
    )(z_flat, cb_padded)
    return codes, loss


def _make_sc_gather(v, d, b):
    b_per_w = b // _NW
    mesh = plsc.VectorSubcoreMesh(core_axis_name="c", subcore_axis_name="s")

    @functools.partial(
        pl.kernel, mesh=mesh,
        out_type=jax.ShapeDtypeStruct((b, d), jnp.float32),
        scratch_types=[
            pltpu.VMEM((b_per_w,), jnp.int32),
            pltpu.VMEM((b_per_w, d), jnp.float32),
            pltpu.SemaphoreType.DMA,
        ],
    )
    def gather(table_hbm, idx_hbm, out_hbm, idx_v, rows_v, sem):
        wid = lax.axis_index("s") * _NC + lax.axis_index("c")
        base = wid * b_per_w
        pltpu.sync_copy(idx_hbm.at[pl.ds(base, b_per_w)], idx_v)
        pltpu.async_copy(table_hbm.at[idx_v], rows_v, sem).wait()
        pltpu.sync_copy(rows_v, out_hbm.at[pl.ds(base, b_per_w)])

    return gather


def kernel(z, codebook):
    b, t, d = z.shape
    k = codebook.shape[0]
    z_flat = z.reshape(-1, d)
    bt = b * t
    n_chunks = -(-k // _K_CHUNK)
    kp = n_chunks * _K_CHUNK
    if kp != k:
        codebook_p = jnp.concatenate(
            [codebook, jnp.full((kp - k, d), _PAD_VAL, jnp.float32)], axis=0)
    else:
        codebook_p = codebook
    inv_norm = (1.0 + _BETA) / float(bt * d)
    codes_col, loss = _distance_argmin(z_flat, codebook_p, inv_norm)
    codes_flat = codes_col.reshape(-1)
    zq_flat = _make_sc_gather(k, d, bt)(codebook, codes_flat)
    z_q_st = zq_flat.reshape(b, t, d)
    codes = codes_flat.reshape(b, t)
    vq_loss = loss[0, 0]
    return z_q_st, codes, vq_loss
